# baseline (speedup 1.0000x reference)
import functools

import jax
import jax.numpy as jnp
from jax import lax
from jax.experimental import pallas as pl
from jax.experimental.pallas import tpu as pltpu
from jax.experimental.pallas import tpu_sc as plsc

N = 10000
E = 320000
D = 128
NC = 2
NS = 16
NW = NC * NS
EPW = E // NW
K = 80
NCHUNK = EPW // K
RPS = 640
NP = NS * RPS
KA = 128
NF = EPW // KA
KR = EPW - NF * KA

_GATHER_DNUMS = lax.GatherDimensionNumbers(
    offset_dims=(), collapsed_slice_dims=(0,), start_index_map=(0,))


def _sc_spmm_body(table_hbm, rows_hbm, cols_hbm, vals_hbm, out_hbm,
                  col_v, row0, row1, val0, val1, rowr, valr, gb0, gb1,
                  acc_sh, sem0, sem1):
    c = lax.axis_index("c")
    s = lax.axis_index("s")
    wid = s * NC + c

    @pl.loop(0, KA)
    def _zfill(r):
        for t in range(8):
            gb0[r, pl.ds(t * 16, 16)] = jnp.zeros((16,), jnp.float32)

    @pl.loop(0, RPS // KA)
    def _zero(i):
        pltpu.sync_copy(gb0, acc_sh.at[pl.ds(s * RPS + i * KA, KA)])

    plsc.subcore_barrier()

    ebase = wid * EPW
    pltpu.sync_copy(cols_hbm.at[pl.ds(ebase, EPW)], col_v)

    def prefetch(i, rowb, valb, gb, sem):
        off = ebase + i * KA
        pltpu.async_copy(rows_hbm.at[pl.ds(off, KA)], rowb, sem)
        pltpu.async_copy(vals_hbm.at[pl.ds(off, KA)], valb, sem)
        pltpu.async_copy(table_hbm.at[col_v.at[pl.ds(i * KA, KA)]], gb, sem)

    def scale(valb, n, gb):
        @pl.loop(0, n // 16)
        def _scale(g):
            vv = valb[pl.ds(g * 16, 16)]
            for l in range(16):
                splat = lax.gather(
                    vv, jnp.full((16, 1), l, jnp.int32),
                    dimension_numbers=_GATHER_DNUMS, slice_sizes=(1,),
                    mode=lax.GatherScatterMode.PROMISE_IN_BOUNDS)
                j = g * 16 + l
                for t in range(8):
                    sl = pl.ds(t * 16, 16)
                    gb[j, sl] = gb[j, sl] * splat

    def process(i, rowb, valb, gb, sem):
        off = ebase + i * KA
        pltpu.make_async_copy(rows_hbm.at[pl.ds(off, KA)], rowb, sem).wait()
        pltpu.make_async_copy(vals_hbm.at[pl.ds(off, KA)], valb, sem).wait()
        pltpu.make_async_copy(table_hbm.at[col_v.at[pl.ds(i * KA, KA)]], gb,
                              sem).wait()
        scale(valb, KA, gb)
        pltpu.sync_copy(gb, acc_sh.at[rowb], add=True)

    prefetch(0, row0, val0, gb0, sem0)

    @pl.loop(0, NF // 2 - 1)
    def _chunk(i):
        prefetch(2 * i + 1, row1, val1, gb1, sem1)
        process(2 * i, row0, val0, gb0, sem0)
        prefetch(2 * i + 2, row0, val0, gb0, sem0)
        process(2 * i + 1, row1, val1, gb1, sem1)

    prefetch(NF - 1, row1, val1, gb1, sem1)
    process(NF - 2, row0, val0, gb0, sem0)

    offr = ebase + NF * KA
    pltpu.async_copy(rows_hbm.at[pl.ds(offr, KR)], rowr, sem0)
    pltpu.async_copy(vals_hbm.at[pl.ds(offr, KR)], valr, sem0)
    pltpu.async_copy(table_hbm.at[col_v.at[pl.ds(NF * KA, KR)]],
                     gb0.at[pl.ds(0, KR)], sem0)

    process(NF - 1, row1, val1, gb1, sem1)

    pltpu.make_async_copy(rows_hbm.at[pl.ds(offr, KR)], rowr, sem0).wait()
    pltpu.make_async_copy(vals_hbm.at[pl.ds(offr, KR)], valr, sem0).wait()
    pltpu.make_async_copy(table_hbm.at[col_v.at[pl.ds(NF * KA, KR)]],
                          gb0.at[pl.ds(0, KR)], sem0).wait()
    scale(valr, KR, gb0)
    pltpu.sync_copy(gb0.at[pl.ds(0, KR)], acc_sh.at[rowr], add=True)

    plsc.subcore_barrier()

    pltpu.sync_copy(acc_sh.at[pl.ds(s * RPS, RPS)],
                    out_hbm.at[pl.ds(c * NP + s * RPS, RPS)])


def _sc_spmm(table, rows, cols, vals):
    mesh = plsc.VectorSubcoreMesh(core_axis_name="c", subcore_axis_name="s")
    kfn = pl.kernel(
        _sc_spmm_body,
        out_type=jax.ShapeDtypeStruct((NC * NP, D), jnp.float32),
        mesh=mesh,
        scratch_types=[
            pltpu.VMEM((EPW,), jnp.int32),
            pltpu.VMEM((KA,), jnp.int32),
            pltpu.VMEM((KA,), jnp.int32),
            pltpu.VMEM((KA,), jnp.float32),
            pltpu.VMEM((KA,), jnp.float32),
            pltpu.VMEM((KR,), jnp.int32),
            pltpu.VMEM((KR,), jnp.float32),
            pltpu.VMEM((KA, D), jnp.float32),
            pltpu.VMEM((KA, D), jnp.float32),
            pltpu.VMEM_SHARED((NP, D), jnp.float32),
            pltpu.SemaphoreType.DMA,
            pltpu.SemaphoreType.DMA,
        ],
    )
    return kfn(table, rows, cols, vals)


NPF = N * D
FPS = NPF // NS


def _sc_densify_body(rows_hbm, cols_hbm, vals_hbm, out_hbm,
                     row_v, col_v, vals_v, flat_v, acc_sh, sem):
    c = lax.axis_index("c")
    s = lax.axis_index("s")
    wid = s * NC + c

    @pl.loop(0, EPW // 16)
    def _zf(r):
        vals_v[pl.ds(r * 16, 16)] = jnp.zeros((16,), jnp.float32)

    @pl.loop(0, FPS // EPW)
    def _zero(i):
        pltpu.sync_copy(vals_v, acc_sh.at[pl.ds(s * FPS + i * EPW, EPW)])

    plsc.subcore_barrier()

    ebase = wid * EPW
    pltpu.sync_copy(rows_hbm.at[pl.ds(ebase, EPW)], row_v)
    pltpu.sync_copy(cols_hbm.at[pl.ds(ebase, EPW)], col_v)
    pltpu.sync_copy(vals_hbm.at[pl.ds(ebase, EPW)], vals_v)

    @pl.loop(0, NCHUNK)
    def _flat(i):
        for g in range(K // 16):
            sl = pl.ds(i * K + g * 16, 16)
            flat_v[i, pl.ds(g * 16, 16)] = row_v[sl] * D + col_v[sl]

    @pl.loop(0, NCHUNK)
    def _scat(i):
        pltpu.async_copy(vals_v.at[pl.ds(i * K, K)], acc_sh.at[flat_v.at[i]],
                         sem, add=True)

    pltpu.make_async_copy(vals_hbm.at[pl.ds(ebase, EPW)], vals_v, sem).wait()

    plsc.subcore_barrier()
    pltpu.sync_copy(acc_sh.at[pl.ds(s * FPS, FPS)],
                    out_hbm.at[pl.ds(c * NPF + s * FPS, FPS)])


def _sc_densify(rows, cols, vals):
    mesh = plsc.VectorSubcoreMesh(core_axis_name="c", subcore_axis_name="s")
    kfn = pl.kernel(
        _sc_densify_body,
        out_type=jax.ShapeDtypeStruct((NC * NPF,), jnp.float32),
        mesh=mesh,
        scratch_types=[
            pltpu.VMEM((EPW,), jnp.int32),
            pltpu.VMEM((EPW,), jnp.int32),
            pltpu.VMEM((EPW,), jnp.float32),
            pltpu.VMEM((NCHUNK, K), jnp.int32),
            pltpu.VMEM_SHARED((NPF,), jnp.float32),
            pltpu.SemaphoreType.DMA,
        ],
    )
    return kfn(rows, cols, vals)


def _mm_relu_body(p_ref, w_ref, b_ref, o_ref):
    s_blk = p_ref[0] + p_ref[1]
    o_ref[...] = jnp.maximum(
        jnp.dot(s_blk, w_ref[...], preferred_element_type=jnp.float32)
        + b_ref[...], 0.0)


def _combine_mm_relu(partial, weight, bias):
    p3 = partial.reshape(NC, N, D)
    return pl.pallas_call(
        _mm_relu_body,
        out_shape=jax.ShapeDtypeStruct((N, D), jnp.float32),
        grid=(N // _BM,),
        in_specs=[
            pl.BlockSpec((NC, _BM, D), lambda i: (0, i, 0)),
            pl.BlockSpec((D, D), lambda i: (0, 0)),
            pl.BlockSpec((1, D), lambda i: (0, 0)),
        ],
        out_specs=pl.BlockSpec((_BM, D), lambda i: (i, 0)),
    )(p3, weight, bias)


def _combine_body(p_ref, o_ref):
    o_ref[...] = p_ref[0] + p_ref[1]


_BM = 2000


def _combine(partial):
    p3 = partial.reshape(NC, NP, D)
    return pl.pallas_call(
        _combine_body,
        out_shape=jax.ShapeDtypeStruct((N, D), jnp.float32),
        grid=(N // _BM,),
        in_specs=[pl.BlockSpec((NC, _BM, D), lambda i: (0, i, 0))],
        out_specs=pl.BlockSpec((_BM, D), lambda i: (i, 0)),
    )(p3)


def kernel(adj_indices, adj_values, feat_indices, feat_values, weight, bias):
    a_rows = adj_indices[0]
    a_cols = adj_indices[1]
    f_rows = feat_indices[0]
    f_cols = feat_indices[1]

    p1 = _sc_densify(f_rows, f_cols, feat_values)
    base = _combine_mm_relu(p1, weight, bias)
    p2 = _sc_spmm(base, a_rows, a_cols, adj_values)
    base = _combine(p2)
    p3 = _sc_spmm(base, a_rows, a_cols, adj_values)
    return _combine(p3)


# speedup vs baseline: 1.0073x; 1.0073x over previous
import functools

import jax
import jax.numpy as jnp
from jax import lax
from jax.experimental import pallas as pl
from jax.experimental.pallas import tpu as pltpu
from jax.experimental.pallas import tpu_sc as plsc

N = 10000
E = 320000
D = 128
NC = 2
NS = 16
NW = NC * NS
EPW = E // NW
K = 80
NCHUNK = EPW // K
RPS = 640
NP = NS * RPS
KA = 128
NF = EPW // KA
KR = EPW - NF * KA

_GATHER_DNUMS = lax.GatherDimensionNumbers(
    offset_dims=(), collapsed_slice_dims=(0,), start_index_map=(0,))


def _sc_spmm_body(table_hbm, rows_hbm, cols_hbm, vals_hbm, out_hbm,
                  col_v, row0, row1, val0, val1, rowr, valr, gb0, gb1,
                  acc_sh, sem0, sem1):
    c = lax.axis_index("c")
    s = lax.axis_index("s")
    wid = s * NC + c

    ebase = wid * EPW
    pltpu.async_copy(cols_hbm.at[pl.ds(ebase, EPW)], col_v, sem1)

    @pl.loop(0, KA)
    def _zfill(r):
        for t in range(8):
            gb0[r, pl.ds(t * 16, 16)] = jnp.zeros((16,), jnp.float32)

    @pl.loop(0, RPS // KA)
    def _zero(i):
        pltpu.sync_copy(gb0, acc_sh.at[pl.ds(s * RPS + i * KA, KA)])

    pltpu.make_async_copy(cols_hbm.at[pl.ds(ebase, EPW)], col_v, sem1).wait()
    plsc.subcore_barrier()

    def prefetch(i, rowb, valb, gb, sem):
        off = ebase + i * KA
        pltpu.async_copy(rows_hbm.at[pl.ds(off, KA)], rowb, sem)
        pltpu.async_copy(vals_hbm.at[pl.ds(off, KA)], valb, sem)
        pltpu.async_copy(table_hbm.at[col_v.at[pl.ds(i * KA, KA)]], gb, sem)

    def scale(valb, n, gb):
        @pl.loop(0, n // 16)
        def _scale(g):
            vv = valb[pl.ds(g * 16, 16)]
            for l in range(16):
                splat = lax.gather(
                    vv, jnp.full((16, 1), l, jnp.int32),
                    dimension_numbers=_GATHER_DNUMS, slice_sizes=(1,),
                    mode=lax.GatherScatterMode.PROMISE_IN_BOUNDS)
                j = g * 16 + l
                for t in range(8):
                    sl = pl.ds(t * 16, 16)
                    gb[j, sl] = gb[j, sl] * splat

    def process(i, rowb, valb, gb, sem):
        off = ebase + i * KA
        pltpu.make_async_copy(rows_hbm.at[pl.ds(off, KA)], rowb, sem).wait()
        pltpu.make_async_copy(vals_hbm.at[pl.ds(off, KA)], valb, sem).wait()
        pltpu.make_async_copy(table_hbm.at[col_v.at[pl.ds(i * KA, KA)]], gb,
                              sem).wait()
        scale(valb, KA, gb)
        pltpu.sync_copy(gb, acc_sh.at[rowb], add=True)

    prefetch(0, row0, val0, gb0, sem0)

    @pl.loop(0, NF // 2 - 1)
    def _chunk(i):
        prefetch(2 * i + 1, row1, val1, gb1, sem1)
        process(2 * i, row0, val0, gb0, sem0)
        prefetch(2 * i + 2, row0, val0, gb0, sem0)
        process(2 * i + 1, row1, val1, gb1, sem1)

    prefetch(NF - 1, row1, val1, gb1, sem1)
    process(NF - 2, row0, val0, gb0, sem0)

    offr = ebase + NF * KA
    pltpu.async_copy(rows_hbm.at[pl.ds(offr, KR)], rowr, sem0)
    pltpu.async_copy(vals_hbm.at[pl.ds(offr, KR)], valr, sem0)
    pltpu.async_copy(table_hbm.at[col_v.at[pl.ds(NF * KA, KR)]],
                     gb0.at[pl.ds(0, KR)], sem0)

    process(NF - 1, row1, val1, gb1, sem1)

    pltpu.make_async_copy(rows_hbm.at[pl.ds(offr, KR)], rowr, sem0).wait()
    pltpu.make_async_copy(vals_hbm.at[pl.ds(offr, KR)], valr, sem0).wait()
    pltpu.make_async_copy(table_hbm.at[col_v.at[pl.ds(NF * KA, KR)]],
                          gb0.at[pl.ds(0, KR)], sem0).wait()
    scale(valr, KR, gb0)
    pltpu.sync_copy(gb0.at[pl.ds(0, KR)], acc_sh.at[rowr], add=True)

    plsc.subcore_barrier()

    pltpu.sync_copy(acc_sh.at[pl.ds(s * RPS, RPS)],
                    out_hbm.at[pl.ds(c * NP + s * RPS, RPS)])


def _sc_spmm(table, rows, cols, vals):
    mesh = plsc.VectorSubcoreMesh(core_axis_name="c", subcore_axis_name="s")
    kfn = pl.kernel(
        _sc_spmm_body,
        out_type=jax.ShapeDtypeStruct((NC * NP, D), jnp.float32),
        mesh=mesh,
        scratch_types=[
            pltpu.VMEM((EPW,), jnp.int32),
            pltpu.VMEM((KA,), jnp.int32),
            pltpu.VMEM((KA,), jnp.int32),
            pltpu.VMEM((KA,), jnp.float32),
            pltpu.VMEM((KA,), jnp.float32),
            pltpu.VMEM((KR,), jnp.int32),
            pltpu.VMEM((KR,), jnp.float32),
            pltpu.VMEM((KA, D), jnp.float32),
            pltpu.VMEM((KA, D), jnp.float32),
            pltpu.VMEM_SHARED((NP, D), jnp.float32),
            pltpu.SemaphoreType.DMA,
            pltpu.SemaphoreType.DMA,
        ],
    )
    return kfn(table, rows, cols, vals)


NPF = N * D
FPS = NPF // NS


def _sc_densify_body(rows_hbm, cols_hbm, vals_hbm, out_hbm,
                     row_v, col_v, vals_v, flat_v, acc_sh, sem):
    c = lax.axis_index("c")
    s = lax.axis_index("s")
    wid = s * NC + c

    @pl.loop(0, EPW // 16)
    def _zf(r):
        vals_v[pl.ds(r * 16, 16)] = jnp.zeros((16,), jnp.float32)

    @pl.loop(0, FPS // EPW)
    def _zero(i):
        pltpu.sync_copy(vals_v, acc_sh.at[pl.ds(s * FPS + i * EPW, EPW)])

    plsc.subcore_barrier()

    ebase = wid * EPW
    pltpu.sync_copy(rows_hbm.at[pl.ds(ebase, EPW)], row_v)
    pltpu.sync_copy(cols_hbm.at[pl.ds(ebase, EPW)], col_v)
    pltpu.sync_copy(vals_hbm.at[pl.ds(ebase, EPW)], vals_v)

    @pl.loop(0, NCHUNK)
    def _flat(i):
        for g in range(K // 16):
            sl = pl.ds(i * K + g * 16, 16)
            flat_v[i, pl.ds(g * 16, 16)] = row_v[sl] * D + col_v[sl]

    @pl.loop(0, NCHUNK)
    def _scat(i):
        pltpu.async_copy(vals_v.at[pl.ds(i * K, K)], acc_sh.at[flat_v.at[i]],
                         sem, add=True)

    pltpu.make_async_copy(vals_hbm.at[pl.ds(ebase, EPW)], vals_v, sem).wait()

    plsc.subcore_barrier()
    pltpu.sync_copy(acc_sh.at[pl.ds(s * FPS, FPS)],
                    out_hbm.at[pl.ds(c * NPF + s * FPS, FPS)])


def _sc_densify(rows, cols, vals):
    mesh = plsc.VectorSubcoreMesh(core_axis_name="c", subcore_axis_name="s")
    kfn = pl.kernel(
        _sc_densify_body,
        out_type=jax.ShapeDtypeStruct((NC * NPF,), jnp.float32),
        mesh=mesh,
        scratch_types=[
            pltpu.VMEM((EPW,), jnp.int32),
            pltpu.VMEM((EPW,), jnp.int32),
            pltpu.VMEM((EPW,), jnp.float32),
            pltpu.VMEM((NCHUNK, K), jnp.int32),
            pltpu.VMEM_SHARED((NPF,), jnp.float32),
            pltpu.SemaphoreType.DMA,
        ],
    )
    return kfn(rows, cols, vals)


def _mm_relu_body(p_ref, w_ref, b_ref, o_ref):
    s_blk = p_ref[0] + p_ref[1]
    o_ref[...] = jnp.maximum(
        jnp.dot(s_blk, w_ref[...], preferred_element_type=jnp.float32)
        + b_ref[...], 0.0)


def _combine_mm_relu(partial, weight, bias):
    p3 = partial.reshape(NC, N, D)
    return pl.pallas_call(
        _mm_relu_body,
        out_shape=jax.ShapeDtypeStruct((N, D), jnp.float32),
        grid=(N // _BM,),
        in_specs=[
            pl.BlockSpec((NC, _BM, D), lambda i: (0, i, 0)),
            pl.BlockSpec((D, D), lambda i: (0, 0)),
            pl.BlockSpec((1, D), lambda i: (0, 0)),
        ],
        out_specs=pl.BlockSpec((_BM, D), lambda i: (i, 0)),
    )(p3, weight, bias)


def _combine_body(p_ref, o_ref):
    o_ref[...] = p_ref[0] + p_ref[1]


_BM = 2000


def _combine(partial):
    p3 = partial.reshape(NC, NP, D)
    return pl.pallas_call(
        _combine_body,
        out_shape=jax.ShapeDtypeStruct((N, D), jnp.float32),
        grid=(N // _BM,),
        in_specs=[pl.BlockSpec((NC, _BM, D), lambda i: (0, i, 0))],
        out_specs=pl.BlockSpec((_BM, D), lambda i: (i, 0)),
    )(p3)


def kernel(adj_indices, adj_values, feat_indices, feat_values, weight, bias):
    a_rows = adj_indices[0]
    a_cols = adj_indices[1]
    f_rows = feat_indices[0]
    f_cols = feat_indices[1]

    p1 = _sc_densify(f_rows, f_cols, feat_values)
    base = _combine_mm_relu(p1, weight, bias)
    p2 = _sc_spmm(base, a_rows, a_cols, adj_values)
    base = _combine(p2)
    p3 = _sc_spmm(base, a_rows, a_cols, adj_values)
    return _combine(p3)
